# Initial kernel scaffold; baseline (speedup 1.0000x reference)
import jax
import jax.numpy as jnp
from jax.experimental import pallas as pl


def kernel(emb, ones_idx, zeros_idx):
    raise NotImplementedError("write your pallas kernel here")


# speedup vs baseline: 3.0460x; 3.0460x over previous
import functools

import numpy as np
import jax
import jax.numpy as jnp
from jax import lax
from jax.experimental import pallas as pl
from jax.experimental.pallas import tpu as pltpu
from jax.experimental.pallas import tpu_sc as plsc

_N_NODES = 10000
_N_EDGES = 320000
_DF = 128
_N_POSSIBLE = _N_NODES * _N_NODES - _N_NODES
_NEG_SCALE = (_N_POSSIBLE - _N_EDGES) / _N_EDGES
_EPS = float(-np.log(1.0 - _N_EDGES / _N_POSSIBLE))

_C = 128
_NCHUNK = _N_EDGES // _C
_NW = 32
_L = 16

_LN2 = 0.6931471805599453
_SQRT2 = 1.4142135623730951


def _log16(y):
    bits = lax.bitcast_convert_type(y, jnp.int32)
    e = ((bits >> 23) & 0xFF) - 127
    m = lax.bitcast_convert_type(
        (bits & 0x007FFFFF) | 0x3F800000, jnp.float32)
    big = m > _SQRT2
    m = jnp.where(big, m * 0.5, m)
    e = jnp.where(big, e + 1, e)
    t = (m - 1.0) / (m + 1.0)
    t2 = t * t
    p = t * (2.0 + t2 * (2.0 / 3.0 + t2 * (2.0 / 5.0 + t2 * (2.0 / 7.0))))
    return e.astype(jnp.float32) * _LN2 + p


def _row_dot_partial(r1_v, r2_v, e):
    acc = r1_v[e, pl.ds(0, _L)] * r2_v[e, pl.ds(0, _L)]
    for k in range(1, _DF // _L):
        acc = acc + r1_v[e, pl.ds(k * _L, _L)] * r2_v[e, pl.ds(k * _L, _L)]
    return acc


_GATHER_DNUMS = lax.GatherDimensionNumbers(
    offset_dims=(), collapsed_slice_dims=(0,), start_index_map=(0,))


def _permute(x, perm):
    return lax.gather(
        x, perm[:, None], dimension_numbers=_GATHER_DNUMS,
        slice_sizes=(1,), mode=lax.GatherScatterMode.PROMISE_IN_BOUNDS)


def _hsum_bcast(x, lane):
    for s in (1, 2, 4, 8):
        x = x + _permute(x, lane ^ s)
    return x


def _build_berpo_sc():
    mesh = plsc.VectorSubcoreMesh(core_axis_name="c", subcore_axis_name="s")
    return functools.partial(
        pl.kernel,
        out_type=[
            jax.ShapeDtypeStruct((_NW, _L), jnp.float32),
            jax.ShapeDtypeStruct((_NW, _L), jnp.float32),
        ],
        mesh=mesh,
        scratch_types=[
            pltpu.VMEM((_C,), jnp.int32),
            pltpu.VMEM((_C,), jnp.int32),
            pltpu.VMEM((_C, _DF), jnp.float32),
            pltpu.VMEM((_C, _DF), jnp.float32),
            pltpu.VMEM((_L,), jnp.float32),
            pltpu.SemaphoreType.DMA,
        ],
    )(_berpo_body)


def _berpo_body(e1, e2, ne1, ne2, emb, out_ones, out_zeros,
                idx1_v, idx2_v, r1_v, r2_v, stage_v, sem):
    cid = lax.axis_index("c")
    sid = lax.axis_index("s")
    wid = sid * 2 + cid
    nt = jnp.where(wid < _NCHUNK % _NW, _NCHUNK // _NW + 1, _NCHUNK // _NW)

    lane = lax.iota(jnp.int32, _L)

    def gather_chunk(ea, eb, chunk):
        base = chunk * _C
        pltpu.sync_copy(ea.at[pl.ds(base, _C)], idx1_v)
        pltpu.sync_copy(eb.at[pl.ds(base, _C)], idx2_v)
        pltpu.async_copy(emb.at[idx1_v], r1_v, sem).wait()
        pltpu.async_copy(emb.at[idx2_v], r2_v, sem).wait()

    def body(t, carry):
        s1, s0 = carry
        chunk = wid + t * _NW

        gather_chunk(e1, e2, chunk)

        def ones_group(g, acc_s1):
            d = jnp.zeros((_L,), jnp.float32)
            for j in range(_L):
                part = _row_dot_partial(r1_v, r2_v, g * _L + j)
                d = jnp.where(lane == j, _hsum_bcast(part, lane), d)
            y = 1.0 - jnp.exp(-_EPS - d)
            return acc_s1 + _log16(y)
        s1 = lax.fori_loop(0, _C // _L, ones_group, s1)

        gather_chunk(ne1, ne2, chunk)

        def zpass(e, acc_s0):
            return acc_s0 + _row_dot_partial(r1_v, r2_v, e)
        s0 = lax.fori_loop(0, _C, zpass, s0)

        return s1, s0

    init = (jnp.zeros((_L,), jnp.float32), jnp.zeros((_L,), jnp.float32))
    s1, s0 = lax.fori_loop(0, nt, body, init)

    stage_v[:] = s1
    pltpu.sync_copy(stage_v, out_ones.at[wid])
    stage_v[:] = s0
    pltpu.sync_copy(stage_v, out_zeros.at[wid])


@functools.cache
def _get_berpo_sc():
    return _build_berpo_sc()


def kernel(emb, ones_idx, zeros_idx):
    e1 = jnp.asarray(ones_idx[:, 0])
    e2 = jnp.asarray(ones_idx[:, 1])
    ne1 = jnp.asarray(zeros_idx[:, 0])
    ne2 = jnp.asarray(zeros_idx[:, 1])
    log_sums, dot_sums = _get_berpo_sc()(e1, e2, ne1, ne2, emb)
    loss_edges = -(jnp.sum(log_sums) / _N_EDGES)
    loss_nonedges = jnp.sum(dot_sums) / _N_EDGES
    return (loss_edges + _NEG_SCALE * loss_nonedges) / (1.0 + _NEG_SCALE)


# speedup vs baseline: 7.0778x; 2.3237x over previous
import functools

import numpy as np
import jax
import jax.numpy as jnp
from jax import lax
from jax.experimental import pallas as pl
from jax.experimental.pallas import tpu as pltpu
from jax.experimental.pallas import tpu_sc as plsc

_N_NODES = 10000
_N_EDGES = 320000
_DF = 128
_N_POSSIBLE = _N_NODES * _N_NODES - _N_NODES
_NEG_SCALE = (_N_POSSIBLE - _N_EDGES) / _N_EDGES
_EPS = float(-np.log(1.0 - _N_EDGES / _N_POSSIBLE))

_C = 128
_NCHUNK = _N_EDGES // _C
_NW = 32
_L = 16
_NT_MAX = -(-_NCHUNK // _NW)
_IDXN = _NT_MAX * _C
_E_PAD = (_NCHUNK + 1) * _C

_LN2 = 0.6931471805599453
_SQRT2 = 1.4142135623730951


def _log16(y):
    bits = lax.bitcast_convert_type(y, jnp.int32)
    e = ((bits >> 23) & 0xFF) - 127
    m = lax.bitcast_convert_type(
        (bits & 0x007FFFFF) | 0x3F800000, jnp.float32)
    big = m > _SQRT2
    m = jnp.where(big, m * 0.5, m)
    e = jnp.where(big, e + 1, e)
    t = (m - 1.0) / (m + 1.0)
    t2 = t * t
    p = t * (2.0 + t2 * (2.0 / 3.0 + t2 * (2.0 / 5.0 + t2 * (2.0 / 7.0))))
    return e.astype(jnp.float32) * _LN2 + p


def _row_dot_partial(r1_v, r2_v, e):
    acc = r1_v[e, pl.ds(0, _L)] * r2_v[e, pl.ds(0, _L)]
    for k in range(1, _DF // _L):
        acc = acc + r1_v[e, pl.ds(k * _L, _L)] * r2_v[e, pl.ds(k * _L, _L)]
    return acc


_GATHER_DNUMS = lax.GatherDimensionNumbers(
    offset_dims=(), collapsed_slice_dims=(0,), start_index_map=(0,))


def _permute(x, perm):
    return lax.gather(
        x, perm[:, None], dimension_numbers=_GATHER_DNUMS,
        slice_sizes=(1,), mode=lax.GatherScatterMode.PROMISE_IN_BOUNDS)


def _merge(a, b, s, lane):
    m = (lane & s) == 0
    return (jnp.where(m, a, b)
            + jnp.where(m, _permute(a, lane ^ s), _permute(b, lane ^ s)))


def _transpose_reduce(parts, lane):
    for s in (1, 2, 4, 8):
        parts = [_merge(parts[2 * i], parts[2 * i + 1], s, lane)
                 for i in range(len(parts) // 2)]
    return parts[0]


def _build_berpo_sc():
    mesh = plsc.VectorSubcoreMesh(core_axis_name="c", subcore_axis_name="s")
    return functools.partial(
        pl.kernel,
        out_type=[
            jax.ShapeDtypeStruct((_NW, _L), jnp.float32),
            jax.ShapeDtypeStruct((_NW, _L), jnp.float32),
        ],
        mesh=mesh,
        scratch_types=[
            pltpu.VMEM((_IDXN,), jnp.int32),
            pltpu.VMEM((_IDXN,), jnp.int32),
            pltpu.VMEM((_IDXN,), jnp.int32),
            pltpu.VMEM((_IDXN,), jnp.int32),
            pltpu.VMEM((_C, _DF), jnp.float32),
            pltpu.VMEM((_C, _DF), jnp.float32),
            pltpu.VMEM((_C, _DF), jnp.float32),
            pltpu.VMEM((_C, _DF), jnp.float32),
            pltpu.VMEM((_L,), jnp.float32),
            pltpu.SemaphoreType.DMA,
            pltpu.SemaphoreType.DMA,
        ],
    )(_berpo_body)


def _berpo_body(e1, e2, ne1, ne2, emb, out_ones, out_zeros,
                io1_v, io2_v, iz1_v, iz2_v,
                ra1_v, ra2_v, rb1_v, rb2_v, stage_v, sem_a, sem_b):
    cid = lax.axis_index("c")
    sid = lax.axis_index("s")
    wid = sid * 2 + cid
    extra = _NCHUNK % _NW
    nt = jnp.where(wid < extra, _NT_MAX, _NT_MAX - 1)
    base = wid * (_NCHUNK // _NW) + jnp.minimum(wid, extra)

    lane = lax.iota(jnp.int32, _L)

    pltpu.sync_copy(e1.at[pl.ds(base * _C, _IDXN)], io1_v)
    pltpu.sync_copy(e2.at[pl.ds(base * _C, _IDXN)], io2_v)
    pltpu.sync_copy(ne1.at[pl.ds(base * _C, _IDXN)], iz1_v)
    pltpu.sync_copy(ne2.at[pl.ds(base * _C, _IDXN)], iz2_v)

    def start_ones(t, d1, d2):
        pltpu.async_copy(emb.at[io1_v.at[pl.ds(t * _C, _C)]], d1, sem_a)
        pltpu.async_copy(emb.at[io2_v.at[pl.ds(t * _C, _C)]], d2, sem_a)

    def start_zeros(t, d1, d2):
        pltpu.async_copy(emb.at[iz1_v.at[pl.ds(t * _C, _C)]], d1, sem_b)
        pltpu.async_copy(emb.at[iz2_v.at[pl.ds(t * _C, _C)]], d2, sem_b)

    def wait(t, idx_v, d1, d2, sem):
        pltpu.make_async_copy(
            emb.at[idx_v.at[pl.ds(t * _C, _C)]], d1, sem).wait()
        pltpu.make_async_copy(
            emb.at[idx_v.at[pl.ds(t * _C, _C)]], d2, sem).wait()

    start_ones(0, ra1_v, ra2_v)

    def body(t, carry):
        s1, s0 = carry

        wait(t, io1_v, ra1_v, ra2_v, sem_a)
        start_zeros(t, rb1_v, rb2_v)

        def ones_group(g, acc_s1):
            parts = [_row_dot_partial(ra1_v, ra2_v, g * _L + j)
                     for j in range(_L)]
            d = _transpose_reduce(parts, lane)
            y = 1.0 - jnp.exp(-_EPS - d)
            return acc_s1 + _log16(y)
        s1 = lax.fori_loop(0, _C // _L, ones_group, s1)

        wait(t, iz1_v, rb1_v, rb2_v, sem_b)

        @pl.when(t + 1 < nt)
        def _():
            start_ones(t + 1, ra1_v, ra2_v)

        def zpass(e, acc_s0):
            return acc_s0 + _row_dot_partial(rb1_v, rb2_v, e)
        s0 = lax.fori_loop(0, _C, zpass, s0, unroll=8)

        return s1, s0

    init = (jnp.zeros((_L,), jnp.float32), jnp.zeros((_L,), jnp.float32))
    s1, s0 = lax.fori_loop(0, nt, body, init)

    stage_v[:] = s1
    pltpu.sync_copy(stage_v, out_ones.at[wid])
    stage_v[:] = s0
    pltpu.sync_copy(stage_v, out_zeros.at[wid])


@functools.cache
def _get_berpo_sc():
    return _build_berpo_sc()


def kernel(emb, ones_idx, zeros_idx):
    pad = _E_PAD - _N_EDGES
    ones_p = jnp.pad(ones_idx, ((0, pad), (0, 0)))
    zeros_p = jnp.pad(zeros_idx, ((0, pad), (0, 0)))
    e1 = jnp.asarray(ones_p[:, 0])
    e2 = jnp.asarray(ones_p[:, 1])
    ne1 = jnp.asarray(zeros_p[:, 0])
    ne2 = jnp.asarray(zeros_p[:, 1])
    log_sums, dot_sums = _get_berpo_sc()(e1, e2, ne1, ne2, emb)
    loss_edges = -(jnp.sum(log_sums) / _N_EDGES)
    loss_nonedges = jnp.sum(dot_sums) / _N_EDGES
    return (loss_edges + _NEG_SCALE * loss_nonedges) / (1.0 + _NEG_SCALE)
